# baseline (speedup 1.0000x reference)
import dataclasses
import functools

import numpy as np
import jax
import jax.numpy as jnp
from jax import lax
from jax.experimental import pallas as pl
from jax.experimental.pallas import tpu as pltpu
from jax.experimental.pallas import tpu_sc as plsc

_N_DIST = 2
_N_THETA = 4
_KAPPA_VM = 2.0
_SIGMA_D = 0.75
_MAX_DIST = 1.5

_NB = 256
_GATHER_WIN = 128
_NHP = 16

_THETAS = np.linspace(-np.pi, np.pi, _N_THETA + 1)[:-1]
_COS_T = np.cos(_THETAS)
_SIN_T = np.sin(_THETAS)
_DISTS = np.linspace(0.0, _MAX_DIST, _N_DIST)


def _sc_gather_rows(table, idx2d):
    V, D = table.shape
    B = idx2d.shape[1]
    mesh = plsc.VectorSubcoreMesh(core_axis_name="c", subcore_axis_name="s")

    @functools.partial(
        pl.kernel,
        mesh=mesh,
        out_type=jax.ShapeDtypeStruct((B, D), table.dtype),
    )
    def gk(x_hbm, i_hbm, o_hbm):
        def body(i_vmem, o_vmem):
            pltpu.sync_copy(x_hbm.at[i_vmem.at[0]], o_vmem)

        pltpu.emit_pipeline(
            body,
            grid=(B // _GATHER_WIN,),
            in_specs=[pl.BlockSpec((1, _GATHER_WIN), lambda i: (0, i))],
            out_specs=[pl.BlockSpec((_GATHER_WIN, D), lambda i: (i, 0))],
            core_axis_name=("c", "s"),
            dimension_semantics=(pltpu.PARALLEL,),
        )(i_hbm, o_hbm)

    return gk(table, idx2d)


def _sc_gather_coords(cx, cy, adjp128):
    n = cx.shape[0]
    nw = 32
    r128 = (n * _NHP // 128) // nw
    mesh = plsc.VectorSubcoreMesh(core_axis_name="c", subcore_axis_name="s")
    cp = pltpu.CompilerParams()
    if "needs_layout_passes" in pltpu.CompilerParams.__dataclass_fields__:
        cp = dataclasses.replace(cp, needs_layout_passes=False)

    @functools.partial(
        pl.kernel,
        mesh=mesh,
        compiler_params=cp,
        out_type=(
            jax.ShapeDtypeStruct((nw * r128, 128), jnp.float32),
            jax.ShapeDtypeStruct((nw * r128, 128), jnp.float32),
        ),
        scratch_types=[
            pltpu.VMEM((n,), jnp.float32),
            pltpu.VMEM((n,), jnp.float32),
            pltpu.VMEM((r128, 128), jnp.int32),
            pltpu.VMEM((r128, 128), jnp.float32),
            pltpu.VMEM((r128, 128), jnp.float32),
        ],
    )
    def ck(cx_hbm, cy_hbm, adj_hbm, ogx_hbm, ogy_hbm, cxv, cyv, idxv, gxv, gyv):
        wid = lax.axis_index("s") * 2 + lax.axis_index("c")
        base = wid * r128
        pltpu.sync_copy(cx_hbm, cxv)
        pltpu.sync_copy(cy_hbm, cyv)
        pltpu.sync_copy(adj_hbm.at[pl.ds(base, r128)], idxv)

        @pl.loop(0, r128)
        def _(i):
            for j in range(128 // 16):
                iv = idxv[i, j * 16:(j + 1) * 16]
                gxv[i, j * 16:(j + 1) * 16] = plsc.load_gather(cxv, [iv])
                gyv[i, j * 16:(j + 1) * 16] = plsc.load_gather(cyv, [iv])

        pltpu.sync_copy(gxv, ogx_hbm.at[pl.ds(base, r128)])
        pltpu.sync_copy(gyv, ogy_hbm.at[pl.ds(base, r128)])

    return ck(cx, cy, adjp128)


def _tc_body(*refs, nb, nh, e, b):
    nh_refs = refs[:9]
    cgx_ref, cgy_ref, xres_ref, w_ref, b_ref, out_ref, wscr_ref = refs[9:]
    npt = _N_DIST * _N_THETA

    cx = cgx_ref[...]
    cy = cgy_ref[...]
    dx = cx - cx[:, 0:1]
    dy = cy - cy[:, 0:1]
    denom = dx * dx + dy * dy
    safe = denom > 1e-12
    dxs = jnp.where(safe, dx, 1.0)
    dys = jnp.where(safe, dy, 0.0)
    dens = jnp.where(safe, denom, 1.0)
    rinv = lax.rsqrt(dens)
    cosphi = dxs * rinv
    sinphi = dys * rinv
    d = jnp.where(safe, dens * rinv, 0.0)
    lane = lax.broadcasted_iota(jnp.int32, (nb, _NHP), 1)
    valid = lane < nh
    wl = []
    for p_i in range(_N_DIST):
        z = (d - _DISTS[p_i]) / _SIGMA_D
        nd = jnp.exp(-0.5 * z * z)
        for t_i in range(_N_THETA):
            a = _KAPPA_VM * (cosphi * _COS_T[t_i] + sinphi * _SIN_T[t_i])
            w = jnp.exp(a) * nd
            w = jnp.where(valid, w, 0.0)
            w = w / (jnp.sum(w, axis=1, keepdims=True) + 1e-10)
            wl.append(w)

    for pt in range(npt):
        wscr_ref[pt] = wl[pt]

    bv = b_ref[...]
    st = 32
    pth = 4

    def sbody(s, carry):
        s0 = s * st
        outs = [bv + xres_ref[b_i, pl.ds(s0, st), :] for b_i in range(b)]
        for pt0 in range(0, npt, pth):
            accs = [[None] * pth for _ in range(b)]
            for h in range(nh):
                xbh = [nh_refs[h][0, pl.ds(s0, st), b_i * e:(b_i + 1) * e]
                       for b_i in range(b)]
                for k in range(pth):
                    wb = jnp.broadcast_to(
                        wscr_ref[pt0 + k, pl.ds(s0, st), h:h + 1], (st, e))
                    for b_i in range(b):
                        t = wb * xbh[b_i]
                        accs[b_i][k] = t if accs[b_i][k] is None \
                            else accs[b_i][k] + t
            for k in range(pth):
                ws = w_ref[(pt0 + k) * e:(pt0 + k + 1) * e, :]
                for b_i in range(b):
                    outs[b_i] = outs[b_i] + jnp.dot(
                        accs[b_i][k], ws, preferred_element_type=jnp.float32)
        for b_i in range(b):
            out_ref[b_i, pl.ds(s0, st), :] = outs[b_i]
        return carry

    lax.fori_loop(0, nb // st, sbody, 0)


def kernel(x, adjc, coordinates, W, b_out):
    b, n, e = x.shape
    nh = adjc.shape[1]
    npt = _N_DIST * _N_THETA

    xcat = jnp.swapaxes(x, 0, 1).reshape(n, b * e)
    idx_hmaj = jnp.swapaxes(adjc, 0, 1).reshape(1, nh * n)
    adjp = jnp.pad(adjc, ((0, 0), (0, _NHP - nh)))
    adjp128 = adjp.reshape(n * _NHP // 128, 128)

    xg = _sc_gather_rows(xcat, idx_hmaj)
    xg3 = xg.reshape(nh, n, b * e)
    cgx, cgy = _sc_gather_coords(coordinates[0], coordinates[1], adjp128)
    cgx = cgx.reshape(n, _NHP)
    cgy = cgy.reshape(n, _NHP)

    body = functools.partial(_tc_body, nb=_NB, nh=nh, e=e, b=b)
    xg_specs = [
        pl.BlockSpec((1, _NB, b * e), functools.partial(
            lambda hk, i: (hk, i, 0), hh))
        for hh in range(nh)
    ]
    out = pl.pallas_call(
        body,
        grid=(n // _NB,),
        in_specs=xg_specs + [
            pl.BlockSpec((_NB, _NHP), lambda i: (i, 0)),
            pl.BlockSpec((_NB, _NHP), lambda i: (i, 0)),
            pl.BlockSpec((b, _NB, e), lambda i: (0, i, 0)),
            pl.BlockSpec((npt * e, e), lambda i: (0, 0)),
            pl.BlockSpec((1, e), lambda i: (0, 0)),
        ],
        out_specs=pl.BlockSpec((b, _NB, e), lambda i: (0, i, 0)),
        out_shape=jax.ShapeDtypeStruct((b, n, e), jnp.float32),
        scratch_shapes=[
            pltpu.VMEM((_N_DIST * _N_THETA, _NB, _NHP), jnp.float32),
        ],
        compiler_params=pltpu.CompilerParams(
            dimension_semantics=("arbitrary",)),
    )(*([xg3] * nh), cgx, cgy, x, W, b_out.reshape(1, e))
    return out


# speedup vs baseline: 1.1200x; 1.1200x over previous
import dataclasses
import functools

import numpy as np
import jax
import jax.numpy as jnp
from jax import lax
from jax.experimental import pallas as pl
from jax.experimental.pallas import tpu as pltpu
from jax.experimental.pallas import tpu_sc as plsc

_N_DIST = 2
_N_THETA = 4
_KAPPA_VM = 2.0
_SIGMA_D = 0.75
_MAX_DIST = 1.5

_NB = 256
_GATHER_WIN = 128
_NHP = 16

_THETAS = np.linspace(-np.pi, np.pi, _N_THETA + 1)[:-1]
_COS_T = np.cos(_THETAS)
_SIN_T = np.sin(_THETAS)
_DISTS = np.linspace(0.0, _MAX_DIST, _N_DIST)


def _sc_gather_rows(table, idx2d):
    V, D = table.shape
    B = idx2d.shape[1]
    mesh = plsc.VectorSubcoreMesh(core_axis_name="c", subcore_axis_name="s")

    @functools.partial(
        pl.kernel,
        mesh=mesh,
        out_type=jax.ShapeDtypeStruct((B, D), table.dtype),
    )
    def gk(x_hbm, i_hbm, o_hbm):
        def body(i_vmem, o_vmem):
            pltpu.sync_copy(x_hbm.at[i_vmem.at[0]], o_vmem)

        pltpu.emit_pipeline(
            body,
            grid=(B // _GATHER_WIN,),
            in_specs=[pl.BlockSpec((1, _GATHER_WIN), lambda i: (0, i))],
            out_specs=[pl.BlockSpec((_GATHER_WIN, D), lambda i: (i, 0))],
            core_axis_name=("c", "s"),
            dimension_semantics=(pltpu.PARALLEL,),
        )(i_hbm, o_hbm)

    return gk(table, idx2d)


def _sc_gather_coords(cx, cy, adjp128):
    n = cx.shape[0]
    nw = 32
    r128 = (n * _NHP // 128) // nw
    mesh = plsc.VectorSubcoreMesh(core_axis_name="c", subcore_axis_name="s")
    cp = pltpu.CompilerParams()
    if "needs_layout_passes" in pltpu.CompilerParams.__dataclass_fields__:
        cp = dataclasses.replace(cp, needs_layout_passes=False)

    @functools.partial(
        pl.kernel,
        mesh=mesh,
        compiler_params=cp,
        out_type=(
            jax.ShapeDtypeStruct((nw * r128, 128), jnp.float32),
            jax.ShapeDtypeStruct((nw * r128, 128), jnp.float32),
        ),
        scratch_types=[
            pltpu.VMEM((n,), jnp.float32),
            pltpu.VMEM((n,), jnp.float32),
            pltpu.VMEM((r128, 128), jnp.int32),
            pltpu.VMEM((r128, 128), jnp.float32),
            pltpu.VMEM((r128, 128), jnp.float32),
        ],
    )
    def ck(cx_hbm, cy_hbm, adj_hbm, ogx_hbm, ogy_hbm, cxv, cyv, idxv, gxv, gyv):
        wid = lax.axis_index("s") * 2 + lax.axis_index("c")
        base = wid * r128
        pltpu.sync_copy(cx_hbm, cxv)
        pltpu.sync_copy(cy_hbm, cyv)
        pltpu.sync_copy(adj_hbm.at[pl.ds(base, r128)], idxv)

        @pl.loop(0, r128)
        def _(i):
            for j in range(128 // 16):
                iv = idxv[i, j * 16:(j + 1) * 16]
                gxv[i, j * 16:(j + 1) * 16] = plsc.load_gather(cxv, [iv])
                gyv[i, j * 16:(j + 1) * 16] = plsc.load_gather(cyv, [iv])

        pltpu.sync_copy(gxv, ogx_hbm.at[pl.ds(base, r128)])
        pltpu.sync_copy(gyv, ogy_hbm.at[pl.ds(base, r128)])

    return ck(cx, cy, adjp128)


def _tc_body(*refs, nb, nh, e, b):
    nh_refs = refs[:9]
    cgx_ref, cgy_ref, xres_ref, w_ref, b_ref, out_ref, wscr_ref = refs[9:]
    npt = _N_DIST * _N_THETA

    cx = cgx_ref[...]
    cy = cgy_ref[...]
    dx = cx - cx[:, 0:1]
    dy = cy - cy[:, 0:1]
    denom = dx * dx + dy * dy
    safe = denom > 1e-12
    dxs = jnp.where(safe, dx, 1.0)
    dys = jnp.where(safe, dy, 0.0)
    dens = jnp.where(safe, denom, 1.0)
    rinv = lax.rsqrt(dens)
    cosphi = dxs * rinv
    sinphi = dys * rinv
    d = jnp.where(safe, dens * rinv, 0.0)
    lane = lax.broadcasted_iota(jnp.int32, (nb, _NHP), 1)
    valid = lane < nh
    wl = []
    for p_i in range(_N_DIST):
        z = (d - _DISTS[p_i]) / _SIGMA_D
        nd = jnp.exp(-0.5 * z * z)
        for t_i in range(_N_THETA):
            a = _KAPPA_VM * (cosphi * _COS_T[t_i] + sinphi * _SIN_T[t_i])
            w = jnp.exp(a) * nd
            w = jnp.where(valid, w, 0.0)
            w = w / (jnp.sum(w, axis=1, keepdims=True) + 1e-10)
            wl.append(w)

    for pt in range(npt):
        wscr_ref[pt] = wl[pt]

    bv = b_ref[...]
    st = 32
    pth = 4

    def sbody(s, carry):
        s0 = s * st
        outs = [bv + xres_ref[b_i, pl.ds(s0, st), :] for b_i in range(b)]
        for pt0 in range(0, npt, pth):
            accs = [[None] * pth for _ in range(b)]
            for h in range(nh):
                xbh = [nh_refs[h][0, pl.ds(s0, st), b_i * e:(b_i + 1) * e]
                       for b_i in range(b)]
                for k in range(pth):
                    wb = jnp.broadcast_to(
                        wscr_ref[pt0 + k, pl.ds(s0, st), h:h + 1], (st, e))
                    for b_i in range(b):
                        t = wb * xbh[b_i]
                        accs[b_i][k] = t if accs[b_i][k] is None \
                            else accs[b_i][k] + t
            for k in range(pth):
                ws = w_ref[(pt0 + k) * e:(pt0 + k + 1) * e, :]
                for b_i in range(b):
                    outs[b_i] = outs[b_i] + jnp.dot(
                        accs[b_i][k], ws, preferred_element_type=jnp.float32)
        for b_i in range(b):
            out_ref[b_i, pl.ds(s0, st), :] = outs[b_i]
        return carry

    lax.fori_loop(0, nb // st, sbody, 0, unroll=2)


def kernel(x, adjc, coordinates, W, b_out):
    b, n, e = x.shape
    nh = adjc.shape[1]
    npt = _N_DIST * _N_THETA

    xcat = jnp.swapaxes(x, 0, 1).reshape(n, b * e)
    idx_hmaj = jnp.swapaxes(adjc, 0, 1).reshape(1, nh * n)
    adjp = jnp.pad(adjc, ((0, 0), (0, _NHP - nh)))
    adjp128 = adjp.reshape(n * _NHP // 128, 128)

    xg = _sc_gather_rows(xcat, idx_hmaj)
    xg3 = xg.reshape(nh, n, b * e)
    cgx, cgy = _sc_gather_coords(coordinates[0], coordinates[1], adjp128)
    cgx = cgx.reshape(n, _NHP)
    cgy = cgy.reshape(n, _NHP)

    body = functools.partial(_tc_body, nb=_NB, nh=nh, e=e, b=b)
    xg_specs = [
        pl.BlockSpec((1, _NB, b * e), functools.partial(
            lambda hk, i: (hk, i, 0), hh))
        for hh in range(nh)
    ]
    out = pl.pallas_call(
        body,
        grid=(n // _NB,),
        in_specs=xg_specs + [
            pl.BlockSpec((_NB, _NHP), lambda i: (i, 0)),
            pl.BlockSpec((_NB, _NHP), lambda i: (i, 0)),
            pl.BlockSpec((b, _NB, e), lambda i: (0, i, 0)),
            pl.BlockSpec((npt * e, e), lambda i: (0, 0)),
            pl.BlockSpec((1, e), lambda i: (0, 0)),
        ],
        out_specs=pl.BlockSpec((b, _NB, e), lambda i: (0, i, 0)),
        out_shape=jax.ShapeDtypeStruct((b, n, e), jnp.float32),
        scratch_shapes=[
            pltpu.VMEM((_N_DIST * _N_THETA, _NB, _NHP), jnp.float32),
        ],
        compiler_params=pltpu.CompilerParams(
            dimension_semantics=("arbitrary",)),
    )(*([xg3] * nh), cgx, cgy, x, W, b_out.reshape(1, e))
    return out


# speedup vs baseline: 1.2301x; 1.0983x over previous
import dataclasses
import functools

import numpy as np
import jax
import jax.numpy as jnp
from jax import lax
from jax.experimental import pallas as pl
from jax.experimental.pallas import tpu as pltpu
from jax.experimental.pallas import tpu_sc as plsc

_N_DIST = 2
_N_THETA = 4
_KAPPA_VM = 2.0
_SIGMA_D = 0.75
_MAX_DIST = 1.5

_NB = 256
_GATHER_WIN = 128
_NHP = 16

_THETAS = np.linspace(-np.pi, np.pi, _N_THETA + 1)[:-1]
_COS_T = np.cos(_THETAS)
_SIN_T = np.sin(_THETAS)
_DISTS = np.linspace(0.0, _MAX_DIST, _N_DIST)


def _sc_gather_rows(table, idx2d):
    V, D = table.shape
    B = idx2d.shape[1]
    mesh = plsc.VectorSubcoreMesh(core_axis_name="c", subcore_axis_name="s")

    @functools.partial(
        pl.kernel,
        mesh=mesh,
        out_type=jax.ShapeDtypeStruct((B, D), table.dtype),
    )
    def gk(x_hbm, i_hbm, o_hbm):
        def body(i_vmem, o_vmem):
            pltpu.sync_copy(x_hbm.at[i_vmem.at[0]], o_vmem)

        pltpu.emit_pipeline(
            body,
            grid=(B // _GATHER_WIN,),
            in_specs=[pl.BlockSpec((1, _GATHER_WIN), lambda i: (0, i))],
            out_specs=[pl.BlockSpec((_GATHER_WIN, D), lambda i: (i, 0))],
            core_axis_name=("c", "s"),
            dimension_semantics=(pltpu.PARALLEL,),
        )(i_hbm, o_hbm)

    return gk(table, idx2d)


def _sc_gather_coords(cx, cy, adjp128):
    n = cx.shape[0]
    nw = 32
    r128 = (n * _NHP // 128) // nw
    mesh = plsc.VectorSubcoreMesh(core_axis_name="c", subcore_axis_name="s")
    cp = pltpu.CompilerParams()
    if "needs_layout_passes" in pltpu.CompilerParams.__dataclass_fields__:
        cp = dataclasses.replace(cp, needs_layout_passes=False)

    @functools.partial(
        pl.kernel,
        mesh=mesh,
        compiler_params=cp,
        out_type=(
            jax.ShapeDtypeStruct((nw * r128, 128), jnp.float32),
            jax.ShapeDtypeStruct((nw * r128, 128), jnp.float32),
        ),
        scratch_types=[
            pltpu.VMEM((n,), jnp.float32),
            pltpu.VMEM((n,), jnp.float32),
            pltpu.VMEM((r128, 128), jnp.int32),
            pltpu.VMEM((r128, 128), jnp.float32),
            pltpu.VMEM((r128, 128), jnp.float32),
        ],
    )
    def ck(cx_hbm, cy_hbm, adj_hbm, ogx_hbm, ogy_hbm, cxv, cyv, idxv, gxv, gyv):
        wid = lax.axis_index("s") * 2 + lax.axis_index("c")
        base = wid * r128
        pltpu.sync_copy(cx_hbm, cxv)
        pltpu.sync_copy(cy_hbm, cyv)
        pltpu.sync_copy(adj_hbm.at[pl.ds(base, r128)], idxv)

        @pl.loop(0, r128)
        def _(i):
            for j in range(128 // 16):
                iv = idxv[i, j * 16:(j + 1) * 16]
                gxv[i, j * 16:(j + 1) * 16] = plsc.load_gather(cxv, [iv])
                gyv[i, j * 16:(j + 1) * 16] = plsc.load_gather(cyv, [iv])

        pltpu.sync_copy(gxv, ogx_hbm.at[pl.ds(base, r128)])
        pltpu.sync_copy(gyv, ogy_hbm.at[pl.ds(base, r128)])

    return ck(cx, cy, adjp128)


def _tc_body(*refs, nb, nh, e, b):
    nh_refs = refs[:9]
    cgx_ref, cgy_ref, xres_ref, w_ref, b_ref, out_ref = refs[9:]
    npt = _N_DIST * _N_THETA

    cx = cgx_ref[...]
    cy = cgy_ref[...]
    dx = cx - cx[:, 0:1]
    dy = cy - cy[:, 0:1]
    denom = dx * dx + dy * dy
    safe = denom > 1e-12
    dxs = jnp.where(safe, dx, 1.0)
    dys = jnp.where(safe, dy, 0.0)
    dens = jnp.where(safe, denom, 1.0)
    rinv = lax.rsqrt(dens)
    cosphi = dxs * rinv
    sinphi = dys * rinv
    d = jnp.where(safe, dens * rinv, 0.0)
    lane = lax.broadcasted_iota(jnp.int32, (nb, _NHP), 1)
    valid = lane < nh
    wl = []
    for p_i in range(_N_DIST):
        z = (d - _DISTS[p_i]) / _SIGMA_D
        nd = jnp.exp(-0.5 * z * z)
        for t_i in range(_N_THETA):
            a = _KAPPA_VM * (cosphi * _COS_T[t_i] + sinphi * _SIN_T[t_i])
            w = jnp.exp(a) * nd
            w = jnp.where(valid, w, 0.0)
            w = w / (jnp.sum(w, axis=1, keepdims=True) + 1e-10)
            wl.append(w)

    bv = b_ref[...]
    st = 32
    pth = 4
    for s in range(nb // st):
        s0 = s * st
        outs = [bv + xres_ref[b_i, s0:s0 + st, :] for b_i in range(b)]
        for pt0 in range(0, npt, pth):
            accs = [[None] * pth for _ in range(b)]
            for h in range(nh):
                xbh = [nh_refs[h][0, s0:s0 + st, b_i * e:(b_i + 1) * e]
                       for b_i in range(b)]
                for k in range(pth):
                    wb = jnp.broadcast_to(
                        wl[pt0 + k][s0:s0 + st, h:h + 1], (st, e))
                    for b_i in range(b):
                        t = wb * xbh[b_i]
                        accs[b_i][k] = t if accs[b_i][k] is None \
                            else accs[b_i][k] + t
            for k in range(pth):
                ws = w_ref[(pt0 + k) * e:(pt0 + k + 1) * e, :]
                for b_i in range(b):
                    outs[b_i] = outs[b_i] + jnp.dot(
                        accs[b_i][k], ws, preferred_element_type=jnp.float32)
        for b_i in range(b):
            out_ref[b_i, s0:s0 + st, :] = outs[b_i]


def kernel(x, adjc, coordinates, W, b_out):
    b, n, e = x.shape
    nh = adjc.shape[1]
    npt = _N_DIST * _N_THETA

    xcat = jnp.swapaxes(x, 0, 1).reshape(n, b * e)
    adjp = jnp.pad(adjc, ((0, 0), (0, _NHP - nh)))
    adjp128 = adjp.reshape(n * _NHP // 128, 128)

    cgx, cgy = _sc_gather_coords(coordinates[0], coordinates[1], adjp128)
    cgx = cgx.reshape(n, _NHP)
    cgy = cgy.reshape(n, _NHP)

    body = functools.partial(_tc_body, nb=_NB, nh=nh, e=e, b=b)
    xg_specs = [
        pl.BlockSpec((1, _NB, b * e), functools.partial(
            lambda hk, i: (hk, i, 0), hh))
        for hh in range(nh)
    ]
    b2 = b_out.reshape(1, e)

    nchunk = 4
    nc = n // nchunk
    outs = []
    for c in range(nchunk):
        sl = slice(c * nc, (c + 1) * nc)
        idx_c = jnp.swapaxes(adjc[sl], 0, 1).reshape(1, nh * nc)
        xg = _sc_gather_rows(xcat, idx_c)
        xg3 = xg.reshape(nh, nc, b * e)
        out_c = pl.pallas_call(
            body,
            grid=(nc // _NB,),
            in_specs=xg_specs + [
                pl.BlockSpec((_NB, _NHP), lambda i: (i, 0)),
                pl.BlockSpec((_NB, _NHP), lambda i: (i, 0)),
                pl.BlockSpec((b, _NB, e), lambda i: (0, i, 0)),
                pl.BlockSpec((npt * e, e), lambda i: (0, 0)),
                pl.BlockSpec((1, e), lambda i: (0, 0)),
            ],
            out_specs=pl.BlockSpec((b, _NB, e), lambda i: (0, i, 0)),
            out_shape=jax.ShapeDtypeStruct((b, nc, e), jnp.float32),
            compiler_params=pltpu.CompilerParams(
                dimension_semantics=("arbitrary",)),
        )(*([xg3] * nh), cgx[sl], cgy[sl], x[:, sl], W, b2)
        outs.append(out_c)
    return jnp.concatenate(outs, axis=1)
